# baseline (speedup 1.0000x reference)
import functools

import jax
import jax.numpy as jnp
from jax import lax
from jax.experimental import pallas as pl
from jax.experimental.pallas import tpu as pltpu
from jax.experimental.pallas import tpu_sc as plsc

N = 100000
D = 128
L = 16
NC, NS = 2, 16
NW = NC * NS
RPW = N // NW
BUF_ROWS = 128
BUF_WORDS = BUF_ROWS * D
FULL_FILLS = RPW // BUF_ROWS
TAIL_ROWS = RPW - FULL_FILLS * BUF_ROWS
TAIL_GROUPS = TAIL_ROWS // L
IDX_BUF = 3136
NB = 4


@functools.cache
def _build():
    mesh = plsc.VectorSubcoreMesh(
        core_axis_name="c", subcore_axis_name="s", num_cores=NC, num_subcores=NS
    )

    @functools.partial(
        pl.kernel,
        out_type=jax.ShapeDtypeStruct((N * D,), jnp.float32),
        mesh=mesh,
        compiler_params=pltpu.CompilerParams(needs_layout_passes=False),
        scratch_types=[
            pltpu.VMEM((4 * D,), jnp.float32),
            pltpu.VMEM((IDX_BUF,), jnp.int32),
            pltpu.VMEM((NB * BUF_WORDS,), jnp.float32),
        ] + [pltpu.SemaphoreType.DMA] * NB,
    )
    def _embed_lookup(table, idx, out, table_v, idx_v, bufring, *ss):
        wid = lax.axis_index("s") * NC + lax.axis_index("c")
        row0 = wid * RPW
        start0 = jnp.minimum((row0 // 8) * 8, N - IDX_BUF)
        off = row0 - start0
        pltpu.sync_copy(table, table_v)
        pltpu.sync_copy(idx.at[pl.ds(start0, IDX_BUF)], idx_v)
        lane = lax.iota(jnp.int32, L)
        obase = row0 * D

        def flush(prev):
            t_prev, wsp = prev
            for c8 in range(D // L):
                bufring[pl.ds(t_prev + c8 * L, L)] = wsp[c8]

        def do_group(boff, wrow, brow, prev):
            v_idx = plsc.load_gather(idx_v, [off + wrow + lane])
            svals = [v_idx[k] * D for k in range(L)]
            for k in range(L):
                ws = []
                if prev is None:
                    for c8 in range(D // L):
                        ws.append(table_v[pl.ds(svals[k] + c8 * L, L)])
                else:
                    t_prev, wsp = prev
                    for c8 in range(D // L):
                        ws.append(table_v[pl.ds(svals[k] + c8 * L, L)])
                        bufring[pl.ds(t_prev + c8 * L, L)] = wsp[c8]
                prev = (boff + (brow + k) * D, ws)
            return prev

        @pl.loop(0, FULL_FILLS, unroll=1)
        def _fill(f):
            b = f % NB
            boff = b * BUF_WORDS
            for bb in range(NB):
                @pl.when(jnp.logical_and(b == bb, f >= NB))
                def _():
                    pltpu.make_async_copy(
                        bufring.at[pl.ds(bb * BUF_WORDS, BUF_WORDS)],
                        out.at[pl.ds(obase + (f - NB) * BUF_WORDS, BUF_WORDS)],
                        ss[bb],
                    ).wait()
            prev = None
            for g in range(BUF_ROWS // L):
                prev = do_group(boff, f * BUF_ROWS + g * L, g * L, prev)
            flush(prev)
            for bb in range(NB):
                @pl.when(b == bb)
                def _():
                    pltpu.async_copy(
                        bufring.at[pl.ds(bb * BUF_WORDS, BUF_WORDS)],
                        out.at[pl.ds(obase + f * BUF_WORDS, BUF_WORDS)],
                        ss[bb],
                    )

        pltpu.make_async_copy(
            bufring.at[pl.ds(0, BUF_WORDS)],
            out.at[pl.ds(obase + (FULL_FILLS - NB) * BUF_WORDS, BUF_WORDS)],
            ss[0],
        ).wait()
        fr = FULL_FILLS * BUF_ROWS
        prev = None
        for g in range(TAIL_GROUPS):
            prev = do_group(0, fr + g * L, g * L, prev)
        prev = do_group(0, fr + TAIL_ROWS - L, TAIL_ROWS - L, prev)
        flush(prev)
        pltpu.async_copy(
            bufring.at[pl.ds(0, TAIL_ROWS * D)],
            out.at[pl.ds(obase + fr * D, TAIL_ROWS * D)],
            ss[0],
        )

        pltpu.make_async_copy(
            bufring.at[pl.ds(0, TAIL_ROWS * D)],
            out.at[pl.ds(obase + fr * D, TAIL_ROWS * D)],
            ss[0],
        ).wait()
        for i in range(1, NB):
            pltpu.make_async_copy(
                bufring.at[pl.ds(i * BUF_WORDS, BUF_WORDS)],
                out.at[pl.ds(obase + (FULL_FILLS - NB + i) * BUF_WORDS, BUF_WORDS)],
                ss[i],
            ).wait()

    return _embed_lookup


def kernel(inputs, embedding):
    idx = inputs.astype(jnp.int32)
    table = embedding.reshape(4 * D)
    out = _build()(table, idx)
    return out.reshape(N, D)


# speedup vs baseline: 1.0285x; 1.0285x over previous
import functools

import jax
import jax.numpy as jnp
from jax import lax
from jax.experimental import pallas as pl
from jax.experimental.pallas import tpu as pltpu
from jax.experimental.pallas import tpu_sc as plsc

N = 100000
D = 128
L = 16
NC, NS = 2, 16
NW = NC * NS
RPW = N // NW
BUF_ROWS = 128
BUF_WORDS = BUF_ROWS * D
FULL_FILLS = RPW // BUF_ROWS
TAIL_ROWS = RPW - FULL_FILLS * BUF_ROWS
TAIL_GROUPS = TAIL_ROWS // L
IDX_BUF = 3136
NB = 4


@functools.cache
def _build():
    mesh = plsc.VectorSubcoreMesh(
        core_axis_name="c", subcore_axis_name="s", num_cores=NC, num_subcores=NS
    )

    @functools.partial(
        pl.kernel,
        out_type=jax.ShapeDtypeStruct((N * D,), jnp.float32),
        mesh=mesh,
        compiler_params=pltpu.CompilerParams(needs_layout_passes=False),
        scratch_types=[
            pltpu.VMEM((4 * D,), jnp.float32),
            pltpu.VMEM((IDX_BUF,), jnp.int32),
            pltpu.VMEM((NB * BUF_WORDS,), jnp.float32),
        ] + [pltpu.SemaphoreType.DMA] * NB,
    )
    def _embed_lookup(table, idx, out, table_v, idx_v, bufring, *ss):
        wid = lax.axis_index("s") * NC + lax.axis_index("c")
        row0 = wid * RPW
        start0 = jnp.minimum((row0 // 8) * 8, N - IDX_BUF)
        off = row0 - start0
        pltpu.sync_copy(table, table_v)
        pltpu.sync_copy(idx.at[pl.ds(start0, IDX_BUF)], idx_v)
        lane = lax.iota(jnp.int32, L)
        obase = row0 * D

        def flush(prev):
            t_prev, wsp = prev
            for c8 in range(D // L):
                bufring[pl.ds(t_prev + c8 * L, L)] = wsp[c8]

        def extract(wrow):
            v_idx = plsc.load_gather(idx_v, [off + wrow + lane])
            return [v_idx[k] * D for k in range(L)]

        def do_group(boff, svals, brow, prev):
            for k in range(L):
                ws = []
                if prev is None:
                    for c8 in range(D // L):
                        ws.append(table_v[pl.ds(svals[k] + c8 * L, L)])
                else:
                    t_prev, wsp = prev
                    for c8 in range(D // L):
                        ws.append(table_v[pl.ds(svals[k] + c8 * L, L)])
                        bufring[pl.ds(t_prev + c8 * L, L)] = wsp[c8]
                prev = (boff + (brow + k) * D, ws)
            return prev

        @pl.loop(0, FULL_FILLS, unroll=1)
        def _fill(f):
            b = f % NB
            boff = b * BUF_WORDS
            for bb in range(NB):
                @pl.when(jnp.logical_and(b == bb, f >= NB))
                def _():
                    pltpu.make_async_copy(
                        bufring.at[pl.ds(bb * BUF_WORDS, BUF_WORDS)],
                        out.at[pl.ds(obase + (f - NB) * BUF_WORDS, BUF_WORDS)],
                        ss[bb],
                    ).wait()
            prev = None
            sv = extract(f * BUF_ROWS)
            for g in range(BUF_ROWS // L):
                sv_next = (
                    extract(f * BUF_ROWS + (g + 1) * L)
                    if g + 1 < BUF_ROWS // L else None
                )
                prev = do_group(boff, sv, g * L, prev)
                sv = sv_next
            flush(prev)
            for bb in range(NB):
                @pl.when(b == bb)
                def _():
                    pltpu.async_copy(
                        bufring.at[pl.ds(bb * BUF_WORDS, BUF_WORDS)],
                        out.at[pl.ds(obase + f * BUF_WORDS, BUF_WORDS)],
                        ss[bb],
                    )

        pltpu.make_async_copy(
            bufring.at[pl.ds(0, BUF_WORDS)],
            out.at[pl.ds(obase + (FULL_FILLS - NB) * BUF_WORDS, BUF_WORDS)],
            ss[0],
        ).wait()
        fr = FULL_FILLS * BUF_ROWS
        tail_wrows = [fr + g * L for g in range(TAIL_GROUPS)] + [fr + TAIL_ROWS - L]
        tail_brows = [g * L for g in range(TAIL_GROUPS)] + [TAIL_ROWS - L]
        prev = None
        sv = extract(tail_wrows[0])
        for i, brow in enumerate(tail_brows):
            sv_next = extract(tail_wrows[i + 1]) if i + 1 < len(tail_wrows) else None
            prev = do_group(0, sv, brow, prev)
            sv = sv_next
        flush(prev)
        pltpu.async_copy(
            bufring.at[pl.ds(0, TAIL_ROWS * D)],
            out.at[pl.ds(obase + fr * D, TAIL_ROWS * D)],
            ss[0],
        )

        pltpu.make_async_copy(
            bufring.at[pl.ds(0, TAIL_ROWS * D)],
            out.at[pl.ds(obase + fr * D, TAIL_ROWS * D)],
            ss[0],
        ).wait()
        for i in range(1, NB):
            pltpu.make_async_copy(
                bufring.at[pl.ds(i * BUF_WORDS, BUF_WORDS)],
                out.at[pl.ds(obase + (FULL_FILLS - NB + i) * BUF_WORDS, BUF_WORDS)],
                ss[i],
            ).wait()

    return _embed_lookup


def kernel(inputs, embedding):
    idx = inputs.astype(jnp.int32)
    table = embedding.reshape(4 * D)
    out = _build()(table, idx)
    return out.reshape(N, D)
